# Initial kernel scaffold; baseline (speedup 1.0000x reference)
import jax
import jax.numpy as jnp
from jax.experimental import pallas as pl


def kernel(noisy_track, clean_track):
    raise NotImplementedError("write your pallas kernel here")


# speedup vs baseline: 2.6841x; 2.6841x over previous
import functools

import jax
import jax.numpy as jnp
import numpy as np
from jax import lax
from jax.experimental import pallas as pl
from jax.experimental.pallas import tpu as pltpu
from jax.experimental.pallas import tpu_sc as plsc

T = 131072
NROWS = 32
LAST = NROWS - 1
NW = 32
RPW = 32
LANE = 128

def _threefry2x32_np(k1, k2, x0, x1):
    k1 = np.uint32(k1); k2 = np.uint32(k2)
    x0 = x0.astype(np.uint32).copy(); x1 = x1.astype(np.uint32).copy()
    ks = [k1, k2, np.uint32(k1 ^ k2 ^ np.uint32(0x1BD11BDA))]
    rot = [np.uint32([13, 15, 26, 6]), np.uint32([17, 29, 16, 24])]

    def rounds(a, b, rs):
        for r in rs:
            a = (a + b).astype(np.uint32)
            b = ((b << r) | (b >> np.uint32(32 - r))).astype(np.uint32)
            b = a ^ b
        return a, b

    x0 = (x0 + ks[0]).astype(np.uint32)
    x1 = (x1 + ks[1]).astype(np.uint32)
    for g, (a, b, c) in enumerate(
            [(1, 2, 1), (2, 0, 2), (0, 1, 3), (1, 2, 4), (2, 0, 5)]):
        x0, x1 = rounds(x0, x1, rot[g % 2])
        x0 = (x0 + ks[a]).astype(np.uint32)
        x1 = (x1 + ks[b] + np.uint32(c)).astype(np.uint32)
    return x0, x1


def _perm_np(seed, n):
    key = (np.uint32(0), np.uint32(seed))
    x = np.arange(n)
    num_rounds = int(np.ceil(3 * np.log(max(1, n)) / np.log(2**32 - 1)))
    for _ in range(num_rounds):
        b1, b2 = _threefry2x32_np(
            key[0], key[1], np.zeros(2, np.uint32),
            np.arange(2, dtype=np.uint32))
        key, sub = (b1[0], b2[0]), (b1[1], b2[1])
        h1, h2 = _threefry2x32_np(
            sub[0], sub[1], np.zeros(n, np.uint32),
            np.arange(n, dtype=np.uint32))
        x = x[np.argsort(h1 ^ h2, kind="stable")]
    return x


_PERM = _perm_np(42, T)
_IDX2_NP = _PERM.astype(np.int32).reshape(T // LANE, LANE)


def _noise_body(n_ref, c_ref, o_ref):
    o_ref[...] = n_ref[...] - c_ref[...]


def _noise_row(noisy_row2, clean_row2):
    return pl.pallas_call(
        _noise_body,
        out_shape=jax.ShapeDtypeStruct((T // LANE, LANE), jnp.float32),
    )(noisy_row2, clean_row2)


def _sc_body(noise1, clean2, idx2, out2, idx_v, buf, gn, sem):
    c = lax.axis_index("c")
    s = lax.axis_index("s")
    wid = s * 2 + c
    r0 = wid * RPW
    pltpu.sync_copy(idx2.at[pl.ds(r0, RPW)], idx_v)
    pltpu.sync_copy(clean2.at[pl.ds(r0, RPW)], buf)
    group = 8
    for g in range(RPW // group):
        descs = [
            pltpu.async_copy(
                noise1.at[idx_v.at[g * group + j]],
                gn.at[g * group + j], sem)
            for j in range(group)
        ]
        for d in descs:
            d.wait()

    def comp(j, carry):
        def inner(k, c2):
            sl = pl.ds(k * 16, 16)
            buf[j, sl] = buf[j, sl] + gn[j, sl]
            return c2
        return lax.fori_loop(0, LANE // 16, inner, carry)
    lax.fori_loop(0, RPW, comp, 0)
    pltpu.sync_copy(buf, out2.at[pl.ds(r0, RPW)])


@functools.cache
def _sc_gather():
    return pl.kernel(
        _sc_body,
        out_type=jax.ShapeDtypeStruct((T // LANE, LANE), jnp.float32),
        mesh=plsc.VectorSubcoreMesh(
            core_axis_name="c", subcore_axis_name="s", num_cores=2,
            num_subcores=16),
        scratch_types=[
            pltpu.VMEM((RPW, LANE), jnp.int32),
            pltpu.VMEM((RPW, LANE), jnp.float32),
            pltpu.VMEM((RPW, LANE), jnp.float32),
            pltpu.SemaphoreType.DMA,
        ],
    )


def _assemble_body(noisy_ref, row_ref, out_ref):
    i = pl.program_id(0)
    rows = i * 8 + lax.broadcasted_iota(jnp.int32, (8, T), 0)
    out_ref[...] = jnp.where(rows == LAST, row_ref[...], noisy_ref[...])


def _assemble(noisy2, new_row2):
    return pl.pallas_call(
        _assemble_body,
        grid=(NROWS // 8,),
        in_specs=[
            pl.BlockSpec((8, T), lambda i: (i, 0)),
            pl.BlockSpec((1, T), lambda i: (0, 0)),
        ],
        out_specs=pl.BlockSpec((8, T), lambda i: (i, 0)),
        out_shape=jax.ShapeDtypeStruct((NROWS, T), jnp.float32),
    )(noisy2, new_row2)


def kernel(noisy_track, clean_track):
    idx2 = jnp.asarray(_IDX2_NP)
    noisy_row2 = noisy_track[-1, -1, :].reshape(T // LANE, LANE)
    clean_row2 = clean_track[-1, -1, :].reshape(T // LANE, LANE)
    noise2 = _noise_row(noisy_row2, clean_row2)
    new_row2 = _sc_gather()(noise2.reshape(T), clean_row2, idx2)
    out2 = _assemble(noisy_track.reshape(NROWS, T), new_row2.reshape(1, T))
    return out2.reshape(16, 2, T), clean_track
